# baseline (speedup 1.0000x reference)
import functools

import jax
import jax.numpy as jnp
from jax import lax
from jax.experimental import pallas as pl
from jax.experimental.pallas import tpu as pltpu
from jax.experimental.pallas import tpu_sc as plsc

NC, NS, L = 2, 16, 16
NW = NC * NS

B, S, D, N = 16, 8192, 256, 1024
H = 32

_MESH = plsc.VectorSubcoreMesh(core_axis_name="c", subcore_axis_name="s")


def _tables_body(emb_ref, w1_ref, b1_ref, ptop_ref, pbot_ref):
    e = emb_ref[...]
    w = w1_ref[...]
    dn = (((0,), (1,)), ((), ()))
    ptop_ref[...] = (
        lax.dot_general(w[:D, :], e, dn, preferred_element_type=jnp.float32)
        + b1_ref[...].reshape(H, 1)
    )
    pbot_ref[...] = lax.dot_general(
        w[D:, :], e, dn, preferred_element_type=jnp.float32
    )


def _tables_tc(emb, W1, b1):
    return pl.pallas_call(
        _tables_body,
        out_shape=(
            jax.ShapeDtypeStruct((H, N), jnp.float32),
            jax.ShapeDtypeStruct((H, N), jnp.float32),
        ),
    )(emb, W1, b1.reshape(1, H))


_EDGE_SLICE = (B * S) // NW
_EDGE_CH = 512


@functools.partial(
    pl.kernel,
    out_type=jax.ShapeDtypeStruct((B * S,), jnp.float32),
    mesh=_MESH,
    compiler_params=pltpu.CompilerParams(needs_layout_passes=False),
    scratch_types=[
        pltpu.VMEM((H, N), jnp.float32),
        pltpu.VMEM((H, N), jnp.float32),
        pltpu.VMEM((48,), jnp.float32),
        pltpu.VMEM((_EDGE_CH,), jnp.int32),
        pltpu.VMEM((_EDGE_CH,), jnp.int32),
        pltpu.VMEM((_EDGE_CH,), jnp.float32),
        pltpu.VMEM((_EDGE_CH,), jnp.float32),
    ],
)
def _edge_sc(ptop_hbm, pbot_hbm, w2_hbm, src_hbm, dst_hbm, vol_hbm, fw_hbm,
             ptop_v, pbot_v, w2_v, src_v, dst_v, vol_v, fw_v):
    wid = lax.axis_index("s") * NC + lax.axis_index("c")
    eb = wid // 2
    ehalf = (wid % 2) * _EDGE_SLICE
    pltpu.sync_copy(ptop_hbm, ptop_v)
    pltpu.sync_copy(pbot_hbm, pbot_v)
    pltpu.sync_copy(w2_hbm, w2_v)
    w2r0 = w2_v[pl.ds(0, L)]
    w2r1 = w2_v[pl.ds(L, L)]
    w2r2 = w2_v[pl.ds(2 * L, L)]
    w2s = [w2r0[j] for j in range(L)] + [w2r1[j] for j in range(L)]
    b2s = w2r2[0]

    def chunk_body(ci, carry):
        cbase = ehalf + ci * _EDGE_CH
        pltpu.sync_copy(src_hbm.at[eb, pl.ds(cbase, _EDGE_CH)], src_v)
        pltpu.sync_copy(dst_hbm.at[eb, pl.ds(cbase, _EDGE_CH)], dst_v)
        pltpu.sync_copy(vol_hbm.at[eb, pl.ds(cbase, _EDGE_CH)], vol_v)

        @plsc.parallel_loop(0, _EDGE_CH // L, step=1)
        def grp_body(g):
            s16 = src_v[pl.ds(g * L, L)]
            d16 = dst_v[pl.ds(g * L, L)]
            v16 = vol_v[pl.ds(g * L, L)]
            acc = jnp.zeros((L,), jnp.float32)
            for j in range(H):
                jj = jnp.full((L,), j, jnp.int32)
                t = plsc.load_gather(ptop_v, [jj, s16])
                u = plsc.load_gather(pbot_v, [jj, d16])
                acc = acc + jnp.maximum(t + u, 0.0) * w2s[j]
            x = acc + b2s
            edge = 1.0 / (1.0 + jnp.exp(-x))
            volw = 1.0 / (1.0 + jnp.exp(v16 * (-1.0 / 1000.0)))
            fw_v[pl.ds(g * L, L)] = edge * volw
        pltpu.sync_copy(fw_v, fw_hbm.at[pl.ds(eb * S + cbase, _EDGE_CH)])
        return carry

    lax.fori_loop(0, _EDGE_SLICE // _EDGE_CH, chunk_body, 0)


_NF_SBLK = 1024


def _nodesum_body(dst_ref, ff_ref, y_ref, acc_ref):
    si = pl.program_id(1)

    @pl.when(si == 0)
    def _():
        acc_ref[...] = jnp.zeros_like(acc_ref)

    dst = dst_ref[...].reshape(1, _NF_SBLK)
    feat = ff_ref[...].reshape(_NF_SBLK, D).astype(jnp.bfloat16)
    iota_n = lax.broadcasted_iota(jnp.int32, (N, _NF_SBLK), 0)
    onehot_t = (iota_n == dst).astype(jnp.bfloat16)
    acc_ref[...] += jnp.dot(onehot_t, feat, preferred_element_type=jnp.float32)

    @pl.when(si == (S // _NF_SBLK) - 1)
    def _():
        x = acc_ref[...]
        ss = jnp.sum(x * x, axis=-1, keepdims=True)
        y_ref[...] = (x / jnp.maximum(jnp.sqrt(ss), 1e-12))[None]


def _nodesum_norm_tc(dst3, flow_features, b0, nb):
    nsi = S // _NF_SBLK
    return pl.pallas_call(
        _nodesum_body,
        grid=(nb, nsi),
        in_specs=[
            pl.BlockSpec(
                (1, 8, _NF_SBLK // 8),
                lambda b, si, _n=nsi, _b0=b0: ((b + _b0) * _n + si, 0, 0),
            ),
            pl.BlockSpec((1, _NF_SBLK, D), lambda b, si, _b0=b0: (b + _b0, si, 0)),
        ],
        out_specs=pl.BlockSpec((1, N, D), lambda b, si: (b, 0, 0)),
        out_shape=jax.ShapeDtypeStruct((nb, N, D), jnp.float32),
        scratch_shapes=[pltpu.VMEM((N, D), jnp.float32)],
    )(dst3, flow_features)


_ADJ_RPT = 64


@functools.partial(
    pl.kernel,
    out_type=jax.ShapeDtypeStruct((B, N, N), jnp.float32),
    mesh=_MESH,
    compiler_params=pltpu.CompilerParams(needs_layout_passes=False),
    scratch_types=[
        pltpu.VMEM((_ADJ_RPT, N), jnp.float32),
        pltpu.VMEM((S,), jnp.int32),
        pltpu.VMEM((S,), jnp.int32),
        pltpu.VMEM((S,), jnp.float32),
        pltpu.VMEM((S,), jnp.int32),
        pltpu.VMEM((S,), jnp.int32),
        pltpu.VMEM((S,), jnp.float32),
        pltpu.SemaphoreType.DMA,
        pltpu.SemaphoreType.DMA,
    ],
)
def _adj_sc(src_hbm, dst_hbm, fw_hbm, adj_hbm, buf_v,
            src0_v, dst0_v, w0_v, src1_v, dst1_v, w1_v, sem_pf, sem_out):
    c = lax.axis_index("c")
    s = lax.axis_index("s")
    row0 = s * _ADJ_RPT
    z16 = jnp.zeros((L,), jnp.float32)
    nb = B // NC
    sets = ((src0_v, dst0_v, w0_v), (src1_v, dst1_v, w1_v))

    def zero_range(r_base, n_rows):
        def zb(i, carry2):
            r = r_base + i // (N // (8 * L))
            cb = (i % (N // (8 * L))) * (8 * L)
            for k in range(8):
                buf_v[r, pl.ds(cb + k * L, L)] = z16
            return carry2

        lax.fori_loop(0, n_rows * N // (8 * L), zb, 0)

    def prefetch(bi, sv, dv, wv):
        b = bi * NC + c
        return (
            pltpu.async_copy(src_hbm.at[b], sv, sem_pf),
            pltpu.async_copy(dst_hbm.at[b], dv, sem_pf),
            pltpu.async_copy(fw_hbm.at[pl.ds(b * S, S)], wv, sem_pf),
        )

    def scan(sv_ref, dv_ref, wv_ref, swap):
        def body(g, carry2):
            for k in range(4):
                off = (4 * g + k) * L
                sv = sv_ref[pl.ds(off, L)]
                dv = dv_ref[pl.ds(off, L)]
                wv = wv_ref[pl.ds(off, L)]
                r = (dv if swap else sv) - row0
                col = sv if swap else dv
                m = plsc.bitcast(r, jnp.uint32) < jnp.uint32(_ADJ_RPT)
                plsc.store_scatter(buf_v, [r, col], wv, mask=m)
            return carry2

        lax.fori_loop(0, S // (4 * L), body, 0)

    for d in prefetch(0, *sets[0]):
        d.wait()
    zero_range(0, _ADJ_RPT)

    for bi in range(nb):
        cur = sets[bi % 2]
        if bi + 1 < nb:
            nxt_descs = prefetch(bi + 1, *sets[(bi + 1) % 2])
        scan(*cur, swap=False)
        scan(*cur, swap=True)
        b = bi * NC + c
        hrows = _ADJ_RPT // 2
        out0 = pltpu.async_copy(
            buf_v.at[pl.ds(0, hrows)],
            adj_hbm.at[b, pl.ds(row0, hrows)],
            sem_out,
        )
        out1 = pltpu.async_copy(
            buf_v.at[pl.ds(hrows, hrows)],
            adj_hbm.at[b, pl.ds(row0 + hrows, hrows)],
            sem_out,
        )
        out0.wait()
        zero_range(0, hrows)
        out1.wait()
        zero_range(hrows, hrows)
        if bi + 1 < nb:
            for d in nxt_descs:
                d.wait()


def kernel(flow_features, src_ips, dst_ips, flow_volumes, emb, W1, b1, W2, b2):
    ptop, pbot = _tables_tc(emb, W1, b1)
    w2pack = jnp.concatenate([W2[:, 0], b2, jnp.zeros((15,), jnp.float32)])
    fw = _edge_sc(ptop, pbot, w2pack, src_ips, dst_ips, flow_volumes)
    dst3 = dst_ips.reshape(B * S // _NF_SBLK, 8, _NF_SBLK // 8)
    q = B // 4
    nf0 = _nodesum_norm_tc(dst3, flow_features, 0, q)
    adjacency = _adj_sc(src_ips, dst_ips, fw)
    nf1 = _nodesum_norm_tc(dst3, flow_features, q, q)
    nf2 = _nodesum_norm_tc(dst3, flow_features, 2 * q, q)
    nf3 = _nodesum_norm_tc(dst3, flow_features, 3 * q, q)
    node_features = jnp.concatenate([nf0, nf1, nf2, nf3], axis=0)
    return node_features, adjacency


# speedup vs baseline: 1.0751x; 1.0751x over previous
import functools

import jax
import jax.numpy as jnp
from jax import lax
from jax.experimental import pallas as pl
from jax.experimental.pallas import tpu as pltpu
from jax.experimental.pallas import tpu_sc as plsc

NC, NS, L = 2, 16, 16
NW = NC * NS

B, S, D, N = 16, 8192, 256, 1024
H = 32

_MESH = plsc.VectorSubcoreMesh(core_axis_name="c", subcore_axis_name="s")


def _tables_body(emb_ref, w1_ref, b1_ref, ptop_ref, pbot_ref):
    e = emb_ref[...]
    w = w1_ref[...]
    dn = (((0,), (1,)), ((), ()))
    ptop_ref[...] = (
        lax.dot_general(w[:D, :], e, dn, preferred_element_type=jnp.float32)
        + b1_ref[...].reshape(H, 1)
    )
    pbot_ref[...] = lax.dot_general(
        w[D:, :], e, dn, preferred_element_type=jnp.float32
    )


def _tables_tc(emb, W1, b1):
    return pl.pallas_call(
        _tables_body,
        out_shape=(
            jax.ShapeDtypeStruct((H, N), jnp.float32),
            jax.ShapeDtypeStruct((H, N), jnp.float32),
        ),
    )(emb, W1, b1.reshape(1, H))


_NF_SBLK = 1024


def _nodesum_body(dst_ref, ff_ref, y_ref, acc_ref):
    si = pl.program_id(1)

    @pl.when(si == 0)
    def _():
        acc_ref[...] = jnp.zeros_like(acc_ref)

    dst = dst_ref[...].reshape(1, _NF_SBLK)
    feat = ff_ref[...].reshape(_NF_SBLK, D).astype(jnp.bfloat16)
    iota_n = lax.broadcasted_iota(jnp.int32, (N, _NF_SBLK), 0)
    onehot_t = (iota_n == dst).astype(jnp.bfloat16)
    acc_ref[...] += jnp.dot(onehot_t, feat, preferred_element_type=jnp.float32)

    @pl.when(si == (S // _NF_SBLK) - 1)
    def _():
        x = acc_ref[...]
        ss = jnp.sum(x * x, axis=-1, keepdims=True)
        y_ref[...] = (x / jnp.maximum(jnp.sqrt(ss), 1e-12))[None]


def _nodesum_norm_tc(dst3, flow_features, b0, nb):
    nsi = S // _NF_SBLK
    return pl.pallas_call(
        _nodesum_body,
        grid=(nb, nsi),
        in_specs=[
            pl.BlockSpec(
                (1, 8, _NF_SBLK // 8),
                lambda b, si, _n=nsi, _b0=b0: ((b + _b0) * _n + si, 0, 0),
            ),
            pl.BlockSpec((1, _NF_SBLK, D), lambda b, si, _b0=b0: (b + _b0, si, 0)),
        ],
        out_specs=pl.BlockSpec((1, N, D), lambda b, si: (b, 0, 0)),
        out_shape=jax.ShapeDtypeStruct((nb, N, D), jnp.float32),
        scratch_shapes=[pltpu.VMEM((N, D), jnp.float32)],
    )(dst3, flow_features)


_EDGE_CH = 512


@functools.partial(
    pl.kernel,
    out_type=jax.ShapeDtypeStruct((B * S,), jnp.float32),
    mesh=_MESH,
    compiler_params=pltpu.CompilerParams(needs_layout_passes=False),
    scratch_types=[
        pltpu.VMEM((H, N), jnp.float32),
        pltpu.VMEM((H, N), jnp.float32),
        pltpu.VMEM((48,), jnp.float32),
        pltpu.VMEM((_EDGE_CH,), jnp.int32),
        pltpu.VMEM((_EDGE_CH,), jnp.int32),
        pltpu.VMEM((_EDGE_CH,), jnp.float32),
        pltpu.VMEM((_EDGE_CH,), jnp.float32),
    ],
)
def _edge_sc(ptop_hbm, pbot_hbm, w2_hbm, src_hbm, dst_hbm, vol_hbm, fw_hbm,
             ptop_v, pbot_v, w2_v, src_v, dst_v, vol_v, fw_v):
    wid = lax.axis_index("s") * NC + lax.axis_index("c")
    eb = wid // 2
    ehalf = (wid % 2) * (S // 2)
    pltpu.sync_copy(ptop_hbm, ptop_v)
    pltpu.sync_copy(pbot_hbm, pbot_v)
    pltpu.sync_copy(w2_hbm, w2_v)
    w2r0 = w2_v[pl.ds(0, L)]
    w2r1 = w2_v[pl.ds(L, L)]
    w2r2 = w2_v[pl.ds(2 * L, L)]
    w2s = [w2r0[j] for j in range(L)] + [w2r1[j] for j in range(L)]
    b2s = w2r2[0]

    def chunk_body(ci, carry):
        cbase = ehalf + ci * _EDGE_CH
        pltpu.sync_copy(src_hbm.at[eb, pl.ds(cbase, _EDGE_CH)], src_v)
        pltpu.sync_copy(dst_hbm.at[eb, pl.ds(cbase, _EDGE_CH)], dst_v)
        pltpu.sync_copy(vol_hbm.at[eb, pl.ds(cbase, _EDGE_CH)], vol_v)

        @plsc.parallel_loop(0, _EDGE_CH // L, step=1)
        def grp_body(g):
            s16 = src_v[pl.ds(g * L, L)]
            d16 = dst_v[pl.ds(g * L, L)]
            v16 = vol_v[pl.ds(g * L, L)]
            acc = jnp.zeros((L,), jnp.float32)
            for j in range(H):
                jj = jnp.full((L,), j, jnp.int32)
                t = plsc.load_gather(ptop_v, [jj, s16])
                u = plsc.load_gather(pbot_v, [jj, d16])
                acc = acc + jnp.maximum(t + u, 0.0) * w2s[j]
            x = acc + b2s
            edge = 1.0 / (1.0 + jnp.exp(-x))
            volw = 1.0 / (1.0 + jnp.exp(v16 * (-1.0 / 1000.0)))
            fw_v[pl.ds(g * L, L)] = edge * volw
        pltpu.sync_copy(fw_v, fw_hbm.at[pl.ds(eb * S + cbase, _EDGE_CH)])
        return carry

    lax.fori_loop(0, (S // 2) // _EDGE_CH, chunk_body, 0)


_ADJ_RPT = 64


@functools.partial(
    pl.kernel,
    out_type=jax.ShapeDtypeStruct((B, N, N), jnp.float32),
    mesh=_MESH,
    compiler_params=pltpu.CompilerParams(needs_layout_passes=False),
    scratch_types=[
        pltpu.VMEM((_ADJ_RPT, N), jnp.float32),
        pltpu.VMEM((S,), jnp.int32),
        pltpu.VMEM((S,), jnp.int32),
        pltpu.VMEM((S,), jnp.float32),
        pltpu.VMEM((S,), jnp.int32),
        pltpu.VMEM((S,), jnp.int32),
        pltpu.VMEM((S,), jnp.float32),
        pltpu.SemaphoreType.DMA,
        pltpu.SemaphoreType.DMA,
    ],
)
def _adj_sc(src_hbm, dst_hbm, fw_hbm, adj_hbm, buf_v,
            src0_v, dst0_v, w0_v, src1_v, dst1_v, w1_v, sem_pf, sem_out):
    c = lax.axis_index("c")
    s = lax.axis_index("s")
    row0 = s * _ADJ_RPT
    z16 = jnp.zeros((L,), jnp.float32)
    nb = B // NC
    sets = ((src0_v, dst0_v, w0_v), (src1_v, dst1_v, w1_v))

    def zero_range(r_base, n_rows):
        def zb(i, carry2):
            r = r_base + i // (N // (8 * L))
            cb = (i % (N // (8 * L))) * (8 * L)
            for k in range(8):
                buf_v[r, pl.ds(cb + k * L, L)] = z16
            return carry2

        lax.fori_loop(0, n_rows * N // (8 * L), zb, 0)

    def prefetch(bi, sv, dv, wv):
        b = bi * NC + c
        return (
            pltpu.async_copy(src_hbm.at[b], sv, sem_pf),
            pltpu.async_copy(dst_hbm.at[b], dv, sem_pf),
            pltpu.async_copy(fw_hbm.at[pl.ds(b * S, S)], wv, sem_pf),
        )

    def scan(sv_ref, dv_ref, wv_ref, swap):
        def body(g, carry2):
            for k in range(4):
                off = (4 * g + k) * L
                sv = sv_ref[pl.ds(off, L)]
                dv = dv_ref[pl.ds(off, L)]
                wv = wv_ref[pl.ds(off, L)]
                r = (dv if swap else sv) - row0
                col = sv if swap else dv
                m = plsc.bitcast(r, jnp.uint32) < jnp.uint32(_ADJ_RPT)
                plsc.store_scatter(buf_v, [r, col], wv, mask=m)
            return carry2

        lax.fori_loop(0, S // (4 * L), body, 0)

    for d in prefetch(0, *sets[0]):
        d.wait()
    zero_range(0, _ADJ_RPT)

    for bi in range(nb):
        cur = sets[bi % 2]
        if bi + 1 < nb:
            nxt_descs = prefetch(bi + 1, *sets[(bi + 1) % 2])
        scan(*cur, swap=False)
        scan(*cur, swap=True)
        b = bi * NC + c
        hrows = _ADJ_RPT // 2
        out0 = pltpu.async_copy(
            buf_v.at[pl.ds(0, hrows)],
            adj_hbm.at[b, pl.ds(row0, hrows)],
            sem_out,
        )
        out1 = pltpu.async_copy(
            buf_v.at[pl.ds(hrows, hrows)],
            adj_hbm.at[b, pl.ds(row0 + hrows, hrows)],
            sem_out,
        )
        out0.wait()
        zero_range(0, hrows)
        out1.wait()
        zero_range(hrows, hrows)
        if bi + 1 < nb:
            for d in nxt_descs:
                d.wait()


def kernel(flow_features, src_ips, dst_ips, flow_volumes, emb, W1, b1, W2, b2):
    ptop, pbot = _tables_tc(emb, W1, b1)
    w2pack = jnp.concatenate([W2[:, 0], b2, jnp.zeros((15,), jnp.float32)])
    fw = _edge_sc(ptop, pbot, w2pack, src_ips, dst_ips, flow_volumes)
    adjacency = _adj_sc(src_ips, dst_ips, fw)
    dst3 = dst_ips.reshape(B * S // _NF_SBLK, 8, _NF_SBLK // 8)
    node_features = _nodesum_norm_tc(dst3, flow_features, 0, B)
    return node_features, adjacency
